# Initial kernel scaffold; baseline (speedup 1.0000x reference)
import functools

import jax
import jax.numpy as jnp
from jax import lax
from jax.experimental import pallas as pl
from jax.experimental.pallas import tpu as pltpu
from jax.experimental.pallas import tpu_sc as plsc

_MAX_LORAS = 8
_INPUT_SIZE = 100000
_OUT_DIM = 16
_B = 16
_L = 4096

_NC = 2
_NS = 16
_NW = _NC * _NS
_ROWS_PER_W = (_B * _L) // _NW
_CHUNK = 128
_NCHUNK = _ROWS_PER_W // _CHUNK
_LANES = 16


def _make_kernel():
    mesh = plsc.VectorSubcoreMesh(core_axis_name="c", subcore_axis_name="s")

    @functools.partial(
        pl.kernel,
        out_type=jax.ShapeDtypeStruct((_B * _L, _OUT_DIM), jnp.float32),
        mesh=mesh,
        scratch_types=[
            pltpu.VMEM((_ROWS_PER_W,), jnp.int32),
            pltpu.VMEM((_ROWS_PER_W, _OUT_DIM), jnp.float32),
            pltpu.VMEM((_B,), jnp.int32),
            pltpu.SemaphoreType.DMA,
        ],
    )
    def lookup(x_hbm, adp_hbm, table_hbm, out_hbm, idx_v, rows_v, adp_v, sem):
        wid = lax.axis_index("s") * _NC + lax.axis_index("c")
        base = wid * _ROWS_PER_W

        pltpu.sync_copy(x_hbm.at[pl.ds(base, _ROWS_PER_W)], idx_v)
        pltpu.sync_copy(adp_hbm, adp_v)

        b = base // _L
        bvec = jnp.full((_LANES,), b, dtype=jnp.int32)
        off = plsc.load_gather(adp_v, [bvec]) * _INPUT_SIZE

        def add_off(i, carry):
            sl = pl.ds(i * _LANES, _LANES)
            idx_v[sl] = idx_v[sl] + off
            return carry

        lax.fori_loop(0, _ROWS_PER_W // _LANES, add_off, 0)

        copies = []
        for j in range(_NCHUNK):
            sl = pl.ds(j * _CHUNK, _CHUNK)
            copies.append(
                pltpu.async_copy(table_hbm.at[idx_v.at[sl]], rows_v.at[sl], sem)
            )
        for c in copies:
            c.wait()

        pltpu.sync_copy(rows_v, out_hbm.at[pl.ds(base, _ROWS_PER_W)])

    return lookup


_lookup = _make_kernel()


@jax.jit
def kernel(x, adapter_ids, weight):
    table = weight.reshape(_MAX_LORAS * _INPUT_SIZE, _OUT_DIM)
    xf = x.reshape(_B * _L).astype(jnp.int32)
    adp = adapter_ids.astype(jnp.int32)
    out = _lookup(xf, adp, table)
    return out.reshape(_B, _L, _OUT_DIM)


# speedup vs baseline: 2.1688x; 2.1688x over previous
import functools

import jax
import jax.numpy as jnp
from jax import lax
from jax.experimental import pallas as pl
from jax.experimental.pallas import tpu as pltpu
from jax.experimental.pallas import tpu_sc as plsc

_MAX_LORAS = 8
_INPUT_SIZE = 100000
_OUT_DIM = 16
_B = 16
_L = 4096

_NC = 2
_NS = 16
_NW = _NC * _NS
_ROWS_PER_W = (_B * _L) // _NW
_CHUNK = 128
_NCHUNK = _ROWS_PER_W // _CHUNK
_LANES = 16


def _make_kernel():
    mesh = plsc.VectorSubcoreMesh(core_axis_name="c", subcore_axis_name="s")

    @functools.partial(
        pl.kernel,
        out_type=jax.ShapeDtypeStruct((_B * _L, _OUT_DIM), jnp.float32),
        mesh=mesh,
        compiler_params=pltpu.CompilerParams(
            needs_layout_passes=False, use_tc_tiling_on_sc=False
        ),
        scratch_types=[
            pltpu.VMEM((_ROWS_PER_W,), jnp.int32),
            pltpu.VMEM((_ROWS_PER_W, _OUT_DIM), jnp.float32),
            pltpu.VMEM((_B,), jnp.int32),
            pltpu.SemaphoreType.DMA,
        ],
    )
    def lookup(x_hbm, adp_hbm, table_hbm, out_hbm, idx_v, rows_v, adp_v, sem):
        wid = lax.axis_index("s") * _NC + lax.axis_index("c")
        base = wid * _ROWS_PER_W

        pltpu.sync_copy(x_hbm.at[pl.ds(base, _ROWS_PER_W)], idx_v)
        pltpu.sync_copy(adp_hbm, adp_v)

        b = base // _L
        adp_vec = adp_v[...]
        lanes = lax.iota(jnp.int32, _LANES)
        sel = jnp.where(lanes == b, adp_vec, 0)
        off = jnp.full((_LANES,), jnp.sum(sel) * _INPUT_SIZE, dtype=jnp.int32)

        def add_off(i, carry):
            sl = pl.ds(i * _LANES, _LANES)
            idx_v[sl] = idx_v[sl] + off
            return carry

        lax.fori_loop(0, _ROWS_PER_W // _LANES, add_off, 0)

        copies = []
        for j in range(_NCHUNK):
            sl = pl.ds(j * _CHUNK, _CHUNK)
            copies.append(
                pltpu.async_copy(table_hbm.at[idx_v.at[sl]], rows_v.at[sl], sem)
            )
        for c in copies:
            c.wait()

        pltpu.sync_copy(rows_v, out_hbm.at[pl.ds(base, _ROWS_PER_W)])

    return lookup


_lookup = _make_kernel()


@jax.jit
def kernel(x, adapter_ids, weight):
    table = weight.reshape(_MAX_LORAS * _INPUT_SIZE, _OUT_DIM)
    xf = x.reshape(_B * _L).astype(jnp.int32)
    adp = adapter_ids.astype(jnp.int32)
    out = _lookup(xf, adp, table)
    return out.reshape(_B, _L, _OUT_DIM)


# speedup vs baseline: 2.2023x; 1.0154x over previous
import functools

import jax
import jax.numpy as jnp
from jax import lax
from jax.experimental import pallas as pl
from jax.experimental.pallas import tpu as pltpu
from jax.experimental.pallas import tpu_sc as plsc

_MAX_LORAS = 8
_INPUT_SIZE = 100000
_OUT_DIM = 16
_B = 16
_L = 4096

_NC = 2
_NS = 16
_NW = _NC * _NS
_ROWS_PER_W = (_B * _L) // _NW
_L_PER_W = _L // 2
_CHUNK = 128
_NCHUNK = _ROWS_PER_W // _CHUNK
_LANES = 16
_NGRP = _ROWS_PER_W // _LANES
_PASS_ROWS = 512
_NPASS = _ROWS_PER_W // _PASS_ROWS
_STREAMS_PER_PASS = _PASS_ROWS // _CHUNK
_GRP_PER_PASS = _PASS_ROWS // _LANES


def _make_kernel():
    mesh = plsc.VectorSubcoreMesh(core_axis_name="c", subcore_axis_name="s")

    @functools.partial(
        pl.kernel,
        out_type=jax.ShapeDtypeStruct((_B, _OUT_DIM, _L), jnp.float32),
        mesh=mesh,
        compiler_params=pltpu.CompilerParams(
            needs_layout_passes=False, use_tc_tiling_on_sc=False
        ),
        scratch_types=[
            pltpu.VMEM((_ROWS_PER_W,), jnp.int32),
            pltpu.VMEM((_ROWS_PER_W,), jnp.int32),
            pltpu.VMEM((_PASS_ROWS, _CHUNK), jnp.float32),
            pltpu.VMEM((_OUT_DIM, _ROWS_PER_W), jnp.float32),
            pltpu.VMEM((_B,), jnp.int32),
            pltpu.SemaphoreType.DMA,
        ],
    )
    def lookup(x_hbm, adp_hbm, table_hbm, out_hbm, idx_v, ridx_v, rows_v,
               out_v, adp_v, sem):
        wid = lax.axis_index("s") * _NC + lax.axis_index("c")
        base = wid * _ROWS_PER_W

        pltpu.sync_copy(x_hbm.at[pl.ds(base, _ROWS_PER_W)], idx_v)
        pltpu.sync_copy(adp_hbm, adp_v)

        b = base // _L
        lanes = lax.iota(jnp.int32, _LANES)
        bvec = jnp.full((_LANES,), 0, jnp.int32) + b
        off = plsc.load_gather(adp_v, [bvec]) * _INPUT_SIZE

        def add_off(i, carry):
            sl = pl.ds(i * _LANES, _LANES)
            g = idx_v[sl] + carry
            idx_v[sl] = g
            ridx_v[sl] = lax.shift_right_logical(g, 3)
            return carry

        lax.fori_loop(0, _NGRP, add_off, off, unroll=4)

        def extract(grp, p0):
            t0 = grp * _LANES
            g16 = idx_v[pl.ds(p0 + t0, _LANES)]
            colbase = (g16 & 7) * _OUT_DIM
            rowvec = t0 + lanes
            outcol = p0 + rowvec
            for d in range(_OUT_DIM):
                vals = plsc.load_gather(rows_v, [rowvec, colbase + d])
                plsc.store_scatter(out_v, [jnp.full((_LANES,), d, jnp.int32),
                                           outcol], vals)
            return p0

        for p in range(_NPASS):
            p0 = p * _PASS_ROWS
            copies = []
            for j in range(_STREAMS_PER_PASS):
                sl = pl.ds(p0 + j * _CHUNK, _CHUNK)
                dst = pl.ds(j * _CHUNK, _CHUNK)
                copies.append(
                    pltpu.async_copy(
                        table_hbm.at[ridx_v.at[sl]], rows_v.at[dst], sem
                    )
                )
            for c in copies:
                c.wait()
            lax.fori_loop(0, _GRP_PER_PASS, extract, p0)

        lbase = (base % _L)
        pltpu.sync_copy(out_v, out_hbm.at[b, :, pl.ds(lbase, _L_PER_W)])

    return lookup


_lookup = _make_kernel()


@jax.jit
def kernel(x, adapter_ids, weight):
    table = weight.reshape(_INPUT_SIZE, 8 * _OUT_DIM)
    xf = x.reshape(_B * _L).astype(jnp.int32)
    adp = adapter_ids.astype(jnp.int32)
    out = _lookup(xf, adp, table)
    return out.transpose(0, 2, 1)


# speedup vs baseline: 6.0235x; 2.7351x over previous
import functools

import jax
import jax.numpy as jnp
from jax import lax
from jax.experimental import pallas as pl
from jax.experimental.pallas import tpu as pltpu
from jax.experimental.pallas import tpu_sc as plsc

_MAX_LORAS = 8
_INPUT_SIZE = 100000
_OUT_DIM = 16
_B = 16
_L = 4096
_TBL = _MAX_LORAS * _OUT_DIM * _INPUT_SIZE

_NC = 2
_NS = 16
_NW = _NC * _NS
_ROWS_PER_W = (_B * _L) // _NW
_L_PER_W = _L // 2
_CHUNK = 128
_NCHUNK = _ROWS_PER_W // _CHUNK
_LANES = 16
_NGRP = _ROWS_PER_W // _LANES
_NSTREAM = _NCHUNK * _OUT_DIM


def _make_kernel():
    mesh = plsc.VectorSubcoreMesh(core_axis_name="c", subcore_axis_name="s")

    @functools.partial(
        pl.kernel,
        out_type=jax.ShapeDtypeStruct((_B, _OUT_DIM, _L), jnp.float32),
        mesh=mesh,
        compiler_params=pltpu.CompilerParams(
            needs_layout_passes=False, use_tc_tiling_on_sc=False
        ),
        scratch_types=[
            pltpu.VMEM((_ROWS_PER_W,), jnp.int32),
            pltpu.VMEM((_NCHUNK, _OUT_DIM, _CHUNK), jnp.int32),
            pltpu.VMEM((_OUT_DIM, _ROWS_PER_W), jnp.float32),
            pltpu.VMEM((_B,), jnp.int32),
            pltpu.SemaphoreType.DMA,
        ],
    )
    def lookup(x_hbm, adp_hbm, table_hbm, out_hbm, idx_v, didx_v, out_v,
               adp_v, sem):
        wid = lax.axis_index("s") * _NC + lax.axis_index("c")
        base = wid * _ROWS_PER_W

        pltpu.sync_copy(x_hbm.at[pl.ds(base, _ROWS_PER_W)], idx_v)
        pltpu.sync_copy(adp_hbm, adp_v)

        b = base // _L
        bvec = jnp.zeros((_LANES,), jnp.int32) + b
        off = plsc.load_gather(adp_v, [bvec]) * (_OUT_DIM * _INPUT_SIZE)

        def add_off(i, carry):
            sl = pl.ds(i * _LANES, _LANES)
            idx_v[sl] = idx_v[sl] + carry
            return carry

        lax.fori_loop(0, _NGRP, add_off, off, unroll=4)

        def fill(k, carry):
            j = lax.shift_right_logical(k, 3)
            k8 = k & 7
            v16 = idx_v[pl.ds(k * _LANES, _LANES)]
            for d in range(_OUT_DIM):
                didx_v[j, d, pl.ds(k8 * _LANES, _LANES)] = (
                    v16 + d * _INPUT_SIZE
                )
            return carry

        lax.fori_loop(0, _NGRP, fill, 0)

        def fire(i, carry):
            d = i & (_OUT_DIM - 1)
            j = lax.shift_right_logical(i, 4)
            pltpu.async_copy(
                table_hbm.at[didx_v.at[j, d]],
                out_v.at[d, pl.ds(j * _CHUNK, _CHUNK)],
                sem,
            )
            return carry

        lax.fori_loop(0, _NSTREAM, fire, 0)

        lbase = base % _L
        dst_view = out_hbm.at[b, :, pl.ds(lbase, _L_PER_W)]
        pltpu.make_async_copy(dst_view, out_v, sem).wait()

        pltpu.sync_copy(out_v, dst_view)

    return lookup


_lookup = _make_kernel()


@jax.jit
def kernel(x, adapter_ids, weight):
    table = weight.transpose(0, 2, 1).reshape(_TBL)
    xf = x.reshape(_B * _L).astype(jnp.int32)
    adp = adapter_ids.astype(jnp.int32)
    out = _lookup(xf, adp, table)
    return out.transpose(0, 2, 1)


# speedup vs baseline: 6.0528x; 1.0049x over previous
import functools

import jax
import jax.numpy as jnp
from jax import lax
from jax.experimental import pallas as pl
from jax.experimental.pallas import tpu as pltpu
from jax.experimental.pallas import tpu_sc as plsc

_MAX_LORAS = 8
_INPUT_SIZE = 100000
_OUT_DIM = 16
_B = 16
_L = 4096
_TBL = _MAX_LORAS * _OUT_DIM * _INPUT_SIZE

_NC = 2
_NS = 16
_NW = _NC * _NS
_ROWS_PER_W = (_B * _L) // _NW
_L_PER_W = _L // 2
_CHUNK = 128
_NCHUNK = _ROWS_PER_W // _CHUNK
_LANES = 16
_NGRP = _ROWS_PER_W // _LANES
_NSTREAM = _NCHUNK * _OUT_DIM


def _make_kernel():
    mesh = plsc.VectorSubcoreMesh(core_axis_name="c", subcore_axis_name="s")

    @functools.partial(
        pl.kernel,
        out_type=jax.ShapeDtypeStruct((_B, _OUT_DIM, _L), jnp.float32),
        mesh=mesh,
        compiler_params=pltpu.CompilerParams(
            needs_layout_passes=False, use_tc_tiling_on_sc=False
        ),
        scratch_types=[
            pltpu.VMEM((_ROWS_PER_W,), jnp.int32),
            pltpu.VMEM((_NCHUNK, _OUT_DIM, _CHUNK), jnp.int32),
            pltpu.VMEM((_OUT_DIM, _ROWS_PER_W), jnp.float32),
            pltpu.VMEM((_B,), jnp.int32),
            pltpu.SemaphoreType.DMA,
        ],
    )
    def lookup(x_hbm, adp_hbm, table_hbm, out_hbm, idx_v, didx_v, out_v,
               adp_v, sem):
        wid = lax.axis_index("s") * _NC + lax.axis_index("c")
        base = wid * _ROWS_PER_W

        pltpu.sync_copy(x_hbm.at[pl.ds(base, _ROWS_PER_W)], idx_v)
        pltpu.sync_copy(adp_hbm, adp_v)

        b = base // _L
        bvec = jnp.zeros((_LANES,), jnp.int32) + b
        off = plsc.load_gather(adp_v, [bvec]) * (_OUT_DIM * _INPUT_SIZE)

        def fill(k, carry):
            j, carry_off = carry
            k8 = k & 7
            v16 = idx_v[pl.ds(j * _CHUNK + k8 * _LANES, _LANES)] + carry_off
            for d in range(_OUT_DIM):
                didx_v[j, d, pl.ds(k8 * _LANES, _LANES)] = (
                    v16 + d * _INPUT_SIZE
                )
            return carry

        for j in range(_NCHUNK):
            lax.fori_loop(0, _CHUNK // _LANES, fill, (j, off))
            for d in range(_OUT_DIM):
                pltpu.async_copy(
                    table_hbm.at[didx_v.at[j, d]],
                    out_v.at[d, pl.ds(j * _CHUNK, _CHUNK)],
                    sem,
                )

        lbase = base % _L
        dst_view = out_hbm.at[b, :, pl.ds(lbase, _L_PER_W)]
        pltpu.make_async_copy(dst_view, out_v, sem).wait()

        pltpu.sync_copy(out_v, dst_view)

    return lookup


_lookup = _make_kernel()


@jax.jit
def kernel(x, adapter_ids, weight):
    table = weight.transpose(0, 2, 1).reshape(_TBL)
    xf = x.reshape(_B * _L).astype(jnp.int32)
    adp = adapter_ids.astype(jnp.int32)
    out = _lookup(xf, adp, table)
    return out.transpose(0, 2, 1)
